# baseline (speedup 1.0000x reference)
import functools

import jax
import jax.numpy as jnp
from jax import lax
from jax.experimental import pallas as pl
from jax.experimental.pallas import tpu as pltpu
from jax.experimental.pallas import tpu_sc as plsc

VOCAB = 1000000
NUM_BUCKETS = 10
EMB = 64
BATCH = 16384

NC, NS, L = 2, 16, 16
NW = NC * NS
BPW = BATCH // NW
GCHUNK = 128


def _body(uid_hbm, age_hbm, table_hbm, agetab_hbm, buckets_hbm, mscale_hbm,
          user_out, age_out, norm_out,
          uid_v, agev_v, idx_v, bins_v, norm_v, urows_v, arows_v,
          buckets_v, mscale_v, sem):
    wid = lax.axis_index("s") * NC + lax.axis_index("c")
    base = wid * BPW

    pltpu.sync_copy(uid_hbm.at[pl.ds(base, BPW)], uid_v)
    pltpu.sync_copy(age_hbm.at[pl.ds(base, BPW)], agev_v)
    pltpu.sync_copy(buckets_hbm, buckets_v)
    pltpu.sync_copy(mscale_hbm, mscale_v)

    mean = mscale_v[0]
    scale = mscale_v[1]

    def compute(g, carry):
        sl = pl.ds(g * L, L)
        uid = uid_v[sl]
        idx_v[sl] = uid + 1
        a = agev_v[sl]
        one = jnp.ones((L,), jnp.int32)
        zero = jnp.zeros((L,), jnp.int32)
        cnt = jnp.where(a >= buckets_v[0], one, zero)
        for j in range(1, NUM_BUCKETS):
            cnt = cnt + jnp.where(a >= buckets_v[j], one, zero)
        bins_v[sl] = cnt
        norm_v[sl] = (a - mean) * scale
        return carry

    lax.fori_loop(0, BPW // L, compute, 0)

    copies = []
    for k in range(BPW // GCHUNK):
        sl = pl.ds(k * GCHUNK, GCHUNK)
        copies.append(pltpu.async_copy(table_hbm.at[idx_v.at[sl]],
                                       urows_v.at[sl], sem))
        copies.append(pltpu.async_copy(agetab_hbm.at[bins_v.at[sl]],
                                       arows_v.at[sl], sem))
    for c in copies:
        c.wait()

    pltpu.sync_copy(urows_v, user_out.at[pl.ds(base, BPW)])
    pltpu.sync_copy(arows_v, age_out.at[pl.ds(base, BPW)])
    pltpu.sync_copy(norm_v, norm_out.at[pl.ds(base, BPW)])


@jax.jit
def _sc_lookup(user_id, age, user_table, age_table, buckets_b, mscale):
    mesh = plsc.VectorSubcoreMesh(core_axis_name="c", subcore_axis_name="s")
    f = pl.kernel(
        _body,
        out_type=[
            jax.ShapeDtypeStruct((BATCH, EMB), jnp.float32),
            jax.ShapeDtypeStruct((BATCH, EMB), jnp.float32),
            jax.ShapeDtypeStruct((BATCH,), jnp.float32),
        ],
        mesh=mesh,
        scratch_types=[
            pltpu.VMEM((BPW,), jnp.int32),
            pltpu.VMEM((BPW,), jnp.float32),
            pltpu.VMEM((BPW,), jnp.int32),
            pltpu.VMEM((BPW,), jnp.int32),
            pltpu.VMEM((BPW,), jnp.float32),
            pltpu.VMEM((BPW, EMB), jnp.float32),
            pltpu.VMEM((BPW, EMB), jnp.float32),
            pltpu.VMEM((NUM_BUCKETS, L), jnp.float32),
            pltpu.VMEM((2, L), jnp.float32),
            pltpu.SemaphoreType.DMA,
        ],
        compiler_params=pltpu.CompilerParams(use_tc_tiling_on_sc=False),
    )
    return f(user_id, age, user_table, age_table, buckets_b, mscale)


def kernel(user_id, age, user_table, age_table, age_buckets, age_mean, age_var):
    buckets_b = jnp.broadcast_to(age_buckets[:, None], (NUM_BUCKETS, L))
    scale = lax.rsqrt(age_var.astype(jnp.float32))
    mscale = jnp.stack([jnp.full((L,), age_mean, jnp.float32),
                        jnp.full((L,), scale, jnp.float32)])
    user_emb, age_emb, norm = _sc_lookup(
        user_id, age, user_table, age_table, buckets_b, mscale)
    return jnp.concatenate([user_emb, age_emb, norm[:, None]], axis=1)


# speedup vs baseline: 1.5282x; 1.5282x over previous
import functools

import jax
import jax.numpy as jnp
from jax import lax
from jax.experimental import pallas as pl
from jax.experimental.pallas import tpu as pltpu
from jax.experimental.pallas import tpu_sc as plsc

VOCAB = 1000000
NUM_BUCKETS = 10
EMB = 64
BATCH = 16384

NC, NS, L = 2, 16, 16
NW = NC * NS
BPW = BATCH // NW
CHUNK = 256


def _body(uid_hbm, age_hbm, table_hbm, agetab_hbm, buckets_hbm, mscale_hbm,
          user_out, age_out, norm_out,
          uid_v, agev_v, bins_v, norm_v, urows_v, arows_v,
          buckets_v, mscale_v, sem, asem):
    wid = lax.axis_index("s") * NC + lax.axis_index("c")
    base = wid * BPW

    pltpu.sync_copy(uid_hbm.at[pl.ds(base, BPW)], uid_v)
    pltpu.sync_copy(age_hbm.at[pl.ds(base, BPW)], agev_v)
    pltpu.sync_copy(buckets_hbm, buckets_v)
    pltpu.sync_copy(mscale_hbm, mscale_v)

    mean = mscale_v[0]
    scale = mscale_v[1]

    def compute(g, carry):
        sl = pl.ds(g * L, L)
        a = agev_v[sl]
        one = jnp.ones((L,), jnp.int32)
        zero = jnp.zeros((L,), jnp.int32)
        cnt = jnp.where(a >= buckets_v[0], one, zero)
        for j in range(1, NUM_BUCKETS):
            cnt = cnt + jnp.where(a >= buckets_v[j], one, zero)
        bins_v[sl] = cnt
        norm_v[sl] = (a - mean) * scale
        return carry

    lax.fori_loop(0, BPW // L, compute, 0)

    for c in range(BPW // CHUNK):
        off = c * CHUNK

        def fetch(g, carry):
            uvec = uid_v[pl.ds(off + g * L, L)]
            bvec = bins_v[pl.ds(off + g * L, L)]
            for l in range(L):
                u = uvec[l] + 1
                pltpu.async_copy(table_hbm.at[pl.ds(u, 1)],
                                 urows_v.at[pl.ds(g * L + l, 1)], sem)
                b = bvec[l]
                pltpu.async_copy(agetab_hbm.at[pl.ds(b, 1)],
                                 arows_v.at[pl.ds(g * L + l, 1)], asem)
            return carry

        lax.fori_loop(0, CHUNK // L, fetch, 0)

        pltpu.make_async_copy(table_hbm.at[pl.ds(0, CHUNK)], urows_v,
                              sem).wait()
        pltpu.make_async_copy(table_hbm.at[pl.ds(0, CHUNK)], arows_v,
                              asem).wait()

        pltpu.sync_copy(urows_v, user_out.at[pl.ds(base + off, CHUNK)])
        pltpu.sync_copy(arows_v, age_out.at[pl.ds(base + off, CHUNK)])

    pltpu.sync_copy(norm_v, norm_out.at[pl.ds(base, BPW)])


@jax.jit
def _sc_lookup(user_id, age, user_table, age_table, buckets_b, mscale):
    mesh = plsc.VectorSubcoreMesh(core_axis_name="c", subcore_axis_name="s")
    f = pl.kernel(
        _body,
        out_type=[
            jax.ShapeDtypeStruct((BATCH, EMB), jnp.float32),
            jax.ShapeDtypeStruct((BATCH, EMB), jnp.float32),
            jax.ShapeDtypeStruct((BATCH,), jnp.float32),
        ],
        mesh=mesh,
        scratch_types=[
            pltpu.VMEM((BPW,), jnp.int32),
            pltpu.VMEM((BPW,), jnp.float32),
            pltpu.VMEM((BPW,), jnp.int32),
            pltpu.VMEM((BPW,), jnp.float32),
            pltpu.VMEM((CHUNK, EMB), jnp.float32),
            pltpu.VMEM((CHUNK, EMB), jnp.float32),
            pltpu.VMEM((NUM_BUCKETS, L), jnp.float32),
            pltpu.VMEM((2, L), jnp.float32),
            pltpu.SemaphoreType.DMA,
            pltpu.SemaphoreType.DMA,
        ],
    )
    return f(user_id, age, user_table, age_table, buckets_b, mscale)


def kernel(user_id, age, user_table, age_table, age_buckets, age_mean, age_var):
    buckets_b = jnp.broadcast_to(age_buckets[:, None], (NUM_BUCKETS, L))
    scale = lax.rsqrt(age_var.astype(jnp.float32))
    mscale = jnp.stack([jnp.full((L,), age_mean, jnp.float32),
                        jnp.full((L,), scale, jnp.float32)])
    user_emb, age_emb, norm = _sc_lookup(
        user_id, age, user_table, age_table, buckets_b, mscale)
    return jnp.concatenate([user_emb, age_emb, norm[:, None]], axis=1)
